# baseline (speedup 1.0000x reference)
import functools

import jax
import jax.numpy as jnp
from jax import lax
from jax.experimental import pallas as pl
from jax.experimental.pallas import tpu as pltpu
from jax.experimental.pallas import tpu_sc as plsc

_NTOKEN = 10000
_NINP = 512
_NHID = 1024
_SEQ = 64
_BATCH = 32
_NG = 3 * _NHID
_B = _SEQ * _BATCH

_NW = 32
_BPW = _B // _NW


def _make_emb_gather():
    mesh = plsc.VectorSubcoreMesh(core_axis_name="c", subcore_axis_name="s")
    out = jax.ShapeDtypeStruct((_B, _NINP), jnp.float32)

    @functools.partial(
        pl.kernel,
        out_type=[out, out],
        mesh=mesh,
        scratch_types=[
            pltpu.VMEM((_BPW,), jnp.int32),
            pltpu.VMEM((_BPW, _NINP), jnp.float32),
            pltpu.VMEM((_BPW, _NINP), jnp.float32),
            pltpu.SemaphoreType.DMA,
            pltpu.SemaphoreType.DMA,
        ],
    )
    def emb_gather(ta, ia, tb, ib, out_a, out_b, idx_v, rows_a, rows_b, sem_a, sem_b):
        wid = lax.axis_index("s") * 2 + lax.axis_index("c")
        base = wid * _BPW
        pltpu.sync_copy(ia.at[pl.ds(base, _BPW)], idx_v)
        cp_a = pltpu.async_copy(ta.at[idx_v], rows_a, sem_a)
        cp_a.wait()
        pltpu.sync_copy(rows_a, out_a.at[pl.ds(base, _BPW)])
        pltpu.sync_copy(ib.at[pl.ds(base, _BPW)], idx_v)
        cp_b = pltpu.async_copy(tb.at[idx_v], rows_b, sem_b)
        cp_b.wait()
        pltpu.sync_copy(rows_b, out_b.at[pl.ds(base, _BPW)])

    return emb_gather


_emb_gather_cache = []


def _emb_gather(ta, ia, tb, ib):
    if not _emb_gather_cache:
        _emb_gather_cache.append(_make_emb_gather())
    return _emb_gather_cache[0](ta, ia, tb, ib)


_DN = (((1,), (1,)), ((), ()))


def _proj2_body(x1_ref, x2_ref, w_ref, b_ref, o_ref):
    acc = lax.dot_general(x1_ref[...], w_ref[:, :_NINP], _DN,
                          preferred_element_type=jnp.float32)
    acc = acc + lax.dot_general(x2_ref[...], w_ref[:, _NINP:], _DN,
                                preferred_element_type=jnp.float32)
    o_ref[...] = (acc + b_ref[...]).astype(jnp.bfloat16)


def _proj_body(x_ref, w_ref, b_ref, o_ref):
    xf = x_ref[...].astype(jnp.float32)
    o_ref[...] = lax.dot_general(xf, w_ref[...], _DN,
                                 preferred_element_type=jnp.float32) + b_ref[...]


_M_BLK = 256


def _input_proj0(emb, emb2, w_ih, b_ih):
    return pl.pallas_call(
        _proj2_body,
        grid=(_B // _M_BLK,),
        in_specs=[
            pl.BlockSpec((_M_BLK, _NINP), lambda i: (i, 0)),
            pl.BlockSpec((_M_BLK, _NINP), lambda i: (i, 0)),
            pl.BlockSpec((_NG, 2 * _NINP), lambda i: (0, 0)),
            pl.BlockSpec((1, _NG), lambda i: (0, 0)),
        ],
        out_specs=pl.BlockSpec((_M_BLK, _NG), lambda i: (i, 0)),
        out_shape=jax.ShapeDtypeStruct((_B, _NG), jnp.bfloat16),
    )(emb, emb2, w_ih, b_ih.reshape(1, _NG))


_N_BLK = 2048
_N_GRID = (_NTOKEN + _N_BLK - 1) // _N_BLK


def _decoder(x, dec_w, dec_b):
    return pl.pallas_call(
        _proj_body,
        grid=(_N_GRID, _B // _M_BLK),
        in_specs=[
            pl.BlockSpec((_M_BLK, _NHID), lambda jn, jm: (jm, 0)),
            pl.BlockSpec((_N_BLK, _NHID), lambda jn, jm: (jn, 0)),
            pl.BlockSpec((1, _N_BLK), lambda jn, jm: (0, jn)),
        ],
        out_specs=pl.BlockSpec((_M_BLK, _N_BLK), lambda jn, jm: (jm, jn)),
        out_shape=jax.ShapeDtypeStruct((_B, _NTOKEN), jnp.float32),
    )(x, dec_w, dec_b.reshape(1, _NTOKEN))


def _gates(gi, gh, h):
    r = jax.nn.sigmoid(gi[:, :_NHID] + gh[:, :_NHID])
    z = jax.nn.sigmoid(gi[:, _NHID:2 * _NHID] + gh[:, _NHID:2 * _NHID])
    n = jnp.tanh(gi[:, 2 * _NHID:] + r * gh[:, 2 * _NHID:])
    return (1.0 - z) * n + z * h


_U = 4
_NS = _SEQ // _U


def _gru2_body(h0i_ref, h1i_ref, gi0_ref, whh0_ref, wih1_ref, whh1_ref,
               decw_ref, bhh0_ref, bih1_ref, bhh1_ref, decb_ref,
               dec_ref, h0l_ref, h1l_ref,
               x1_ref, yd_ref, h0_ref, h1_ref):
    s = pl.program_id(0)

    @pl.when(s == 0)
    def _init():
        h0_ref[...] = h0i_ref[...]
        h1_ref[...] = h1i_ref[...]

    x1_prev = x1_ref[...].reshape(_U * _BATCH, _NHID)
    ydec = yd_ref[...].reshape(_U * _BATCH, _NHID)
    h0 = h0_ref[...]
    h1 = h1_ref[...]

    bf = jnp.bfloat16

    gi1c = jnp.dot(x1_prev.astype(bf), wih1_ref[...],
                   preferred_element_type=jnp.float32) + bih1_ref[...]

    logits = jnp.dot(ydec, decw_ref[...],
                     preferred_element_type=jnp.float32) + decb_ref[...]

    h0outs = []
    for u in range(_U):
        gh0 = jnp.dot(h0.astype(bf), whh0_ref[...],
                      preferred_element_type=jnp.float32) + bhh0_ref[...]
        h0 = _gates(gi0_ref[u], gh0, h0)
        h0outs.append(h0)

    h1outs = []
    for u in range(_U):
        gh1 = jnp.dot(h1.astype(bf), whh1_ref[...],
                      preferred_element_type=jnp.float32) + bhh1_ref[...]
        h1 = _gates(gi1c[u * _BATCH:(u + 1) * _BATCH], gh1, h1)
        h1outs.append(h1)

    @pl.when(s < _NS)
    def _commit0():
        h0_ref[...] = h0
        for u in range(_U):
            x1_ref[u] = h0outs[u]
        h0l_ref[...] = h0

    @pl.when((s > 0) & (s <= _NS))
    def _commit1():
        h1_ref[...] = h1
        for u in range(_U):
            yd_ref[u] = h1outs[u].astype(bf)
        h1l_ref[...] = h1

    @pl.when(s >= 2)
    def _commit_dec():
        dec_ref[...] = logits


def _gru2_dec(gi0, h0, h1, w_hh0, b_hh0, w_ih1, b_ih1, w_hh1, b_hh1,
              dec_w, dec_b):
    cvmem = lambda: pltpu.VMEM((_BATCH, _NHID), jnp.float32)
    wspec = pl.BlockSpec((_NHID, _NG), lambda t: (0, 0))
    bspec = pl.BlockSpec((1, _NG), lambda t: (0, 0))
    hspec = pl.BlockSpec((_BATCH, _NHID), lambda t: (0, 0))
    pl_call = pl.pallas_call(
        _gru2_body,
        grid=(_NS + 2,),
        in_specs=[
            hspec,
            hspec,
            pl.BlockSpec((_U, _BATCH, _NG),
                         lambda t: (jnp.clip(t, 0, _NS - 1), 0, 0)),
            wspec,
            wspec,
            wspec,
            pl.BlockSpec((_NHID, _NTOKEN), lambda t: (0, 0)),
            bspec,
            bspec,
            bspec,
            pl.BlockSpec((1, _NTOKEN), lambda t: (0, 0)),
        ],
        out_specs=[
            pl.BlockSpec((_U * _BATCH, _NTOKEN),
                         lambda t: (jnp.clip(t - 2, 0, _NS - 1), 0)),
            hspec,
            hspec,
        ],
        out_shape=[
            jax.ShapeDtypeStruct((_B, _NTOKEN), jnp.float32),
            jax.ShapeDtypeStruct((_BATCH, _NHID), jnp.float32),
            jax.ShapeDtypeStruct((_BATCH, _NHID), jnp.float32),
        ],
        scratch_shapes=[pltpu.VMEM((_U, _BATCH, _NHID), jnp.float32),
                        pltpu.VMEM((_U, _BATCH, _NHID), jnp.bfloat16),
                        cvmem(), cvmem()],
        compiler_params=pltpu.CompilerParams(
            vmem_limit_bytes=100 * 1024 * 1024),
    )
    bf = jnp.bfloat16
    return pl_call(h0, h1, gi0, w_hh0.T.astype(bf), w_ih1.T.astype(bf),
                   w_hh1.T.astype(bf), dec_w.T.astype(bf),
                   b_hh0.reshape(1, _NG), b_ih1.reshape(1, _NG),
                   b_hh1.reshape(1, _NG), dec_b.reshape(1, _NTOKEN))


def kernel(input, hidden, extra_notes, enc_w, enc_lyr_w, W_ih0, W_hh0, b_ih0,
           b_hh0, W_ih1, W_hh1, b_ih1, b_hh1, dec_w, dec_b):
    half = input.shape[0] // 2
    idx_a = input[:half].reshape(_B)
    idx_b = input[half:].reshape(_B)

    emb, emb2 = _emb_gather(enc_w, idx_a, enc_lyr_w, idx_b)

    gi0 = _input_proj0(emb, emb2, W_ih0, b_ih0)
    decoded, h0f, h1f = _gru2_dec(gi0.reshape(_SEQ, _BATCH, _NG), hidden[0],
                                  hidden[1], W_hh0, b_hh0, W_ih1, b_ih1,
                                  W_hh1, b_hh1, dec_w, dec_b)

    decoded = decoded.reshape(_SEQ, _BATCH, _NTOKEN)
    hidden_out = jnp.stack([h0f, h1f], axis=0)
    return decoded, hidden_out


# speedup vs baseline: 1.0472x; 1.0472x over previous
import functools

import jax
import jax.numpy as jnp
from jax import lax
from jax.experimental import pallas as pl
from jax.experimental.pallas import tpu as pltpu
from jax.experimental.pallas import tpu_sc as plsc

_NTOKEN = 10000
_NINP = 512
_NHID = 1024
_SEQ = 64
_BATCH = 32
_NG = 3 * _NHID
_B = _SEQ * _BATCH

_NW = 32
_BPW = _B // _NW


def _make_emb_gather():
    mesh = plsc.VectorSubcoreMesh(core_axis_name="c", subcore_axis_name="s")
    out = jax.ShapeDtypeStruct((_B, _NINP), jnp.float32)

    @functools.partial(
        pl.kernel,
        out_type=[out, out],
        mesh=mesh,
        scratch_types=[
            pltpu.VMEM((_BPW,), jnp.int32),
            pltpu.VMEM((_BPW,), jnp.int32),
            pltpu.VMEM((_BPW, _NINP), jnp.float32),
            pltpu.VMEM((_BPW, _NINP), jnp.float32),
            pltpu.SemaphoreType.DMA,
            pltpu.SemaphoreType.DMA,
        ],
    )
    def emb_gather(ta, ia, tb, ib, out_a, out_b, idx_a, idx_b, rows_a, rows_b,
                   sem_a, sem_b):
        wid = lax.axis_index("s") * 2 + lax.axis_index("c")
        base = wid * _BPW
        pltpu.sync_copy(ia.at[pl.ds(base, _BPW)], idx_a)
        pltpu.sync_copy(ib.at[pl.ds(base, _BPW)], idx_b)
        cp_a = pltpu.async_copy(ta.at[idx_a], rows_a, sem_a)
        cp_b = pltpu.async_copy(tb.at[idx_b], rows_b, sem_b)
        cp_a.wait()
        pltpu.sync_copy(rows_a, out_a.at[pl.ds(base, _BPW)])
        cp_b.wait()
        pltpu.sync_copy(rows_b, out_b.at[pl.ds(base, _BPW)])

    return emb_gather


_emb_gather_cache = []


def _emb_gather(ta, ia, tb, ib):
    if not _emb_gather_cache:
        _emb_gather_cache.append(_make_emb_gather())
    return _emb_gather_cache[0](ta, ia, tb, ib)


_DN = (((1,), (1,)), ((), ()))


def _proj2_body(x1_ref, x2_ref, w_ref, b_ref, o_ref):
    bf = jnp.bfloat16
    acc = jnp.dot(x1_ref[...].astype(bf), w_ref[:_NINP],
                  preferred_element_type=jnp.float32)
    acc = acc + jnp.dot(x2_ref[...].astype(bf), w_ref[_NINP:],
                        preferred_element_type=jnp.float32)
    o_ref[...] = (acc + b_ref[...]).astype(bf)


def _proj_body(x_ref, w_ref, b_ref, o_ref):
    xf = x_ref[...].astype(jnp.float32)
    o_ref[...] = lax.dot_general(xf, w_ref[...], _DN,
                                 preferred_element_type=jnp.float32) + b_ref[...]


_M_BLK = 256
_MP_BLK = 512


def _input_proj0(emb, emb2, w_ih, b_ih):
    return pl.pallas_call(
        _proj2_body,
        grid=(_B // _MP_BLK,),
        in_specs=[
            pl.BlockSpec((_MP_BLK, _NINP), lambda i: (i, 0)),
            pl.BlockSpec((_MP_BLK, _NINP), lambda i: (i, 0)),
            pl.BlockSpec((2 * _NINP, _NG), lambda i: (0, 0)),
            pl.BlockSpec((1, _NG), lambda i: (0, 0)),
        ],
        out_specs=pl.BlockSpec((_MP_BLK, _NG), lambda i: (i, 0)),
        out_shape=jax.ShapeDtypeStruct((_B, _NG), jnp.bfloat16),
    )(emb, emb2, w_ih.T.astype(jnp.bfloat16), b_ih.reshape(1, _NG))


_N_BLK = 2048
_N_GRID = (_NTOKEN + _N_BLK - 1) // _N_BLK


def _decoder(x, dec_w, dec_b):
    return pl.pallas_call(
        _proj_body,
        grid=(_N_GRID, _B // _M_BLK),
        in_specs=[
            pl.BlockSpec((_M_BLK, _NHID), lambda jn, jm: (jm, 0)),
            pl.BlockSpec((_N_BLK, _NHID), lambda jn, jm: (jn, 0)),
            pl.BlockSpec((1, _N_BLK), lambda jn, jm: (0, jn)),
        ],
        out_specs=pl.BlockSpec((_M_BLK, _N_BLK), lambda jn, jm: (jm, jn)),
        out_shape=jax.ShapeDtypeStruct((_B, _NTOKEN), jnp.float32),
    )(x, dec_w, dec_b.reshape(1, _NTOKEN))


def _gates(gi, gh, h):
    r = jax.nn.sigmoid(gi[:, :_NHID] + gh[:, :_NHID])
    z = jax.nn.sigmoid(gi[:, _NHID:2 * _NHID] + gh[:, _NHID:2 * _NHID])
    n = jnp.tanh(gi[:, 2 * _NHID:] + r * gh[:, 2 * _NHID:])
    return (1.0 - z) * n + z * h


_U = 4
_NS = _SEQ // _U


def _gru2_body(h0i_ref, h1i_ref, gi0_ref, whh0_ref, wih1_ref, whh1_ref,
               bhh0_ref, bih1_ref, bhh1_ref, y_ref, h0l_ref,
               x1_ref, h0_ref, h1_ref):
    s = pl.program_id(0)

    @pl.when(s == 0)
    def _init():
        h0_ref[...] = h0i_ref[...]
        h1_ref[...] = h1i_ref[...]

    x1_prev = x1_ref[...].reshape(_U * _BATCH, _NHID)
    h0 = h0_ref[...]
    h1 = h1_ref[...]

    bf = jnp.bfloat16

    gi1c = jnp.dot(x1_prev.astype(bf), wih1_ref[...],
                   preferred_element_type=jnp.float32) + bih1_ref[...]

    h0outs = []
    for u in range(_U):
        gh0 = jnp.dot(h0.astype(bf), whh0_ref[...],
                      preferred_element_type=jnp.float32) + bhh0_ref[...]
        h0 = _gates(gi0_ref[u], gh0, h0)
        h0outs.append(h0)

    h1outs = []
    for u in range(_U):
        gh1 = jnp.dot(h1.astype(bf), whh1_ref[...],
                      preferred_element_type=jnp.float32) + bhh1_ref[...]
        h1 = _gates(gi1c[u * _BATCH:(u + 1) * _BATCH], gh1, h1)
        h1outs.append(h1)

    @pl.when(s < _NS)
    def _commit0():
        h0_ref[...] = h0
        for u in range(_U):
            x1_ref[u] = h0outs[u]
        h0l_ref[...] = h0

    @pl.when(s > 0)
    def _commit1():
        h1_ref[...] = h1
        for u in range(_U):
            y_ref[u] = h1outs[u].astype(bf)


def _gru2(gi0, h0, h1, w_hh0, b_hh0, w_ih1, b_ih1, w_hh1, b_hh1):
    cvmem = lambda: pltpu.VMEM((_BATCH, _NHID), jnp.float32)
    pl_call = pl.pallas_call(
        _gru2_body,
        grid=(_NS + 1,),
        in_specs=[
            pl.BlockSpec((_BATCH, _NHID), lambda t: (0, 0)),
            pl.BlockSpec((_BATCH, _NHID), lambda t: (0, 0)),
            pl.BlockSpec((_U, _BATCH, _NG),
                         lambda t: (jnp.minimum(t, _NS - 1), 0, 0)),
            pl.BlockSpec((_NHID, _NG), lambda t: (0, 0)),
            pl.BlockSpec((_NHID, _NG), lambda t: (0, 0)),
            pl.BlockSpec((_NHID, _NG), lambda t: (0, 0)),
            pl.BlockSpec((1, _NG), lambda t: (0, 0)),
            pl.BlockSpec((1, _NG), lambda t: (0, 0)),
            pl.BlockSpec((1, _NG), lambda t: (0, 0)),
        ],
        out_specs=[
            pl.BlockSpec((_U, _BATCH, _NHID),
                         lambda t: (jnp.maximum(t - 1, 0), 0, 0)),
            pl.BlockSpec((_BATCH, _NHID), lambda t: (0, 0)),
        ],
        out_shape=[
            jax.ShapeDtypeStruct((_SEQ, _BATCH, _NHID), jnp.bfloat16),
            jax.ShapeDtypeStruct((_BATCH, _NHID), jnp.float32),
        ],
        scratch_shapes=[pltpu.VMEM((_U, _BATCH, _NHID), jnp.float32),
                        cvmem(), cvmem()],
    )
    bf = jnp.bfloat16
    return pl_call(h0, h1, gi0, w_hh0.T.astype(bf), w_ih1.T.astype(bf),
                   w_hh1.T.astype(bf), b_hh0.reshape(1, _NG),
                   b_ih1.reshape(1, _NG), b_hh1.reshape(1, _NG))


def kernel(input, hidden, extra_notes, enc_w, enc_lyr_w, W_ih0, W_hh0, b_ih0,
           b_hh0, W_ih1, W_hh1, b_ih1, b_hh1, dec_w, dec_b):
    half = input.shape[0] // 2
    idx_a = input[:half].reshape(_B)
    idx_b = input[half:].reshape(_B)

    emb, emb2 = _emb_gather(enc_w, idx_a, enc_lyr_w, idx_b)

    gi0 = _input_proj0(emb, emb2, W_ih0, b_ih0)
    y1, h0f = _gru2(gi0.reshape(_SEQ, _BATCH, _NG), hidden[0], hidden[1],
                    W_hh0, b_hh0, W_ih1, b_ih1, W_hh1, b_hh1)

    decoded = _decoder(y1.reshape(_B, _NHID), dec_w, dec_b)
    decoded = decoded.reshape(_SEQ, _BATCH, _NTOKEN)
    hidden_out = jnp.stack([h0f, y1[-1].astype(jnp.float32)], axis=0)
    return decoded, hidden_out


# speedup vs baseline: 1.0759x; 1.0274x over previous
import functools

import jax
import jax.numpy as jnp
from jax import lax
from jax.experimental import pallas as pl
from jax.experimental.pallas import tpu as pltpu
from jax.experimental.pallas import tpu_sc as plsc

_NTOKEN = 10000
_NINP = 512
_NHID = 1024
_SEQ = 64
_BATCH = 32
_NG = 3 * _NHID
_B = _SEQ * _BATCH

_NW = 32
_BPW = _B // _NW


def _make_emb_gather():
    mesh = plsc.VectorSubcoreMesh(core_axis_name="c", subcore_axis_name="s")
    out = jax.ShapeDtypeStruct((_B, _NINP), jnp.float32)

    @functools.partial(
        pl.kernel,
        out_type=[out, out],
        mesh=mesh,
        scratch_types=[
            pltpu.VMEM((_BPW,), jnp.int32),
            pltpu.VMEM((_BPW,), jnp.int32),
            pltpu.VMEM((_BPW, _NINP), jnp.float32),
            pltpu.VMEM((_BPW, _NINP), jnp.float32),
            pltpu.SemaphoreType.DMA,
            pltpu.SemaphoreType.DMA,
        ],
    )
    def emb_gather(ta, ia, tb, ib, out_a, out_b, idx_a, idx_b, rows_a, rows_b,
                   sem_a, sem_b):
        wid = lax.axis_index("s") * 2 + lax.axis_index("c")
        base = wid * _BPW
        pltpu.sync_copy(ia.at[pl.ds(base, _BPW)], idx_a)
        pltpu.sync_copy(ib.at[pl.ds(base, _BPW)], idx_b)
        cp_a = pltpu.async_copy(ta.at[idx_a], rows_a, sem_a)
        cp_b = pltpu.async_copy(tb.at[idx_b], rows_b, sem_b)
        cp_a.wait()
        pltpu.sync_copy(rows_a, out_a.at[pl.ds(base, _BPW)])
        cp_b.wait()
        pltpu.sync_copy(rows_b, out_b.at[pl.ds(base, _BPW)])

    return emb_gather


_emb_gather_cache = []


def _emb_gather(ta, ia, tb, ib):
    if not _emb_gather_cache:
        _emb_gather_cache.append(_make_emb_gather())
    return _emb_gather_cache[0](ta, ia, tb, ib)


_DN = (((1,), (1,)), ((), ()))


def _proj2_body(x1_ref, x2_ref, w_ref, b_ref, o_ref):
    acc = lax.dot_general(x1_ref[...], w_ref[:, :_NINP], _DN,
                          preferred_element_type=jnp.float32)
    acc = acc + lax.dot_general(x2_ref[...], w_ref[:, _NINP:], _DN,
                                preferred_element_type=jnp.float32)
    o_ref[...] = (acc + b_ref[...]).astype(jnp.bfloat16)


def _proj_body(x_ref, w_ref, b_ref, o_ref):
    xf = x_ref[...].astype(jnp.float32)
    o_ref[...] = lax.dot_general(xf, w_ref[...], _DN,
                                 preferred_element_type=jnp.float32) + b_ref[...]


_M_BLK = 256
_MP_BLK = 512


def _input_proj0(emb, emb2, w_ih, b_ih):
    return pl.pallas_call(
        _proj2_body,
        grid=(_B // _MP_BLK,),
        in_specs=[
            pl.BlockSpec((_MP_BLK, _NINP), lambda i: (i, 0)),
            pl.BlockSpec((_MP_BLK, _NINP), lambda i: (i, 0)),
            pl.BlockSpec((_NG, 2 * _NINP), lambda i: (0, 0)),
            pl.BlockSpec((1, _NG), lambda i: (0, 0)),
        ],
        out_specs=pl.BlockSpec((_MP_BLK, _NG), lambda i: (i, 0)),
        out_shape=jax.ShapeDtypeStruct((_B, _NG), jnp.bfloat16),
    )(emb, emb2, w_ih, b_ih.reshape(1, _NG))


_N_BLK = 2048
_N_GRID = (_NTOKEN + _N_BLK - 1) // _N_BLK


def _decoder(x, dec_w, dec_b):
    return pl.pallas_call(
        _proj_body,
        grid=(_N_GRID, _B // _M_BLK),
        in_specs=[
            pl.BlockSpec((_M_BLK, _NHID), lambda jn, jm: (jm, 0)),
            pl.BlockSpec((_N_BLK, _NHID), lambda jn, jm: (jn, 0)),
            pl.BlockSpec((1, _N_BLK), lambda jn, jm: (0, jn)),
        ],
        out_specs=pl.BlockSpec((_M_BLK, _N_BLK), lambda jn, jm: (jm, jn)),
        out_shape=jax.ShapeDtypeStruct((_B, _NTOKEN), jnp.float32),
    )(x, dec_w, dec_b.reshape(1, _NTOKEN))


def _gates(gi, gh, h):
    r = jax.nn.sigmoid(gi[:, :_NHID] + gh[:, :_NHID])
    z = jax.nn.sigmoid(gi[:, _NHID:2 * _NHID] + gh[:, _NHID:2 * _NHID])
    n = jnp.tanh(gi[:, 2 * _NHID:] + r * gh[:, 2 * _NHID:])
    return (1.0 - z) * n + z * h


_U = 4
_NS = _SEQ // _U


def _gru2_body(h0i_ref, h1i_ref, gi0_ref, whh0_ref, wih1_ref, whh1_ref,
               bhh0_ref, bih1_ref, bhh1_ref, y_ref, h0l_ref,
               x1_ref, h0_ref, h1_ref):
    s = pl.program_id(0)

    @pl.when(s == 0)
    def _init():
        h0_ref[...] = h0i_ref[...]
        h1_ref[...] = h1i_ref[...]

    x1_prev = x1_ref[...].reshape(_U * _BATCH, _NHID)
    h0 = h0_ref[...]
    h1 = h1_ref[...]

    bf = jnp.bfloat16

    gi1c = jnp.dot(x1_prev.astype(bf), wih1_ref[...],
                   preferred_element_type=jnp.float32) + bih1_ref[...]

    h0outs = []
    for u in range(_U):
        gh0 = jnp.dot(h0.astype(bf), whh0_ref[...],
                      preferred_element_type=jnp.float32) + bhh0_ref[...]
        h0 = _gates(gi0_ref[u], gh0, h0)
        h0outs.append(h0)

    h1outs = []
    for u in range(_U):
        gh1 = jnp.dot(h1.astype(bf), whh1_ref[...],
                      preferred_element_type=jnp.float32) + bhh1_ref[...]
        h1 = _gates(gi1c[u * _BATCH:(u + 1) * _BATCH], gh1, h1)
        h1outs.append(h1)

    @pl.when(s < _NS)
    def _commit0():
        h0_ref[...] = h0
        for u in range(_U):
            x1_ref[u] = h0outs[u]
        h0l_ref[...] = h0

    @pl.when(s > 0)
    def _commit1():
        h1_ref[...] = h1
        for u in range(_U):
            y_ref[u] = h1outs[u].astype(bf)


def _gru2(gi0, h0, h1, w_hh0, b_hh0, w_ih1, b_ih1, w_hh1, b_hh1):
    cvmem = lambda: pltpu.VMEM((_BATCH, _NHID), jnp.float32)
    pl_call = pl.pallas_call(
        _gru2_body,
        grid=(_NS + 1,),
        in_specs=[
            pl.BlockSpec((_BATCH, _NHID), lambda t: (0, 0)),
            pl.BlockSpec((_BATCH, _NHID), lambda t: (0, 0)),
            pl.BlockSpec((_U, _BATCH, _NG),
                         lambda t: (jnp.minimum(t, _NS - 1), 0, 0)),
            pl.BlockSpec((_NHID, _NG), lambda t: (0, 0)),
            pl.BlockSpec((_NHID, _NG), lambda t: (0, 0)),
            pl.BlockSpec((_NHID, _NG), lambda t: (0, 0)),
            pl.BlockSpec((1, _NG), lambda t: (0, 0)),
            pl.BlockSpec((1, _NG), lambda t: (0, 0)),
            pl.BlockSpec((1, _NG), lambda t: (0, 0)),
        ],
        out_specs=[
            pl.BlockSpec((_U, _BATCH, _NHID),
                         lambda t: (jnp.maximum(t - 1, 0), 0, 0)),
            pl.BlockSpec((_BATCH, _NHID), lambda t: (0, 0)),
        ],
        out_shape=[
            jax.ShapeDtypeStruct((_SEQ, _BATCH, _NHID), jnp.bfloat16),
            jax.ShapeDtypeStruct((_BATCH, _NHID), jnp.float32),
        ],
        scratch_shapes=[pltpu.VMEM((_U, _BATCH, _NHID), jnp.float32),
                        cvmem(), cvmem()],
    )
    bf = jnp.bfloat16
    return pl_call(h0, h1, gi0, w_hh0.T.astype(bf), w_ih1.T.astype(bf),
                   w_hh1.T.astype(bf), b_hh0.reshape(1, _NG),
                   b_ih1.reshape(1, _NG), b_hh1.reshape(1, _NG))


def kernel(input, hidden, extra_notes, enc_w, enc_lyr_w, W_ih0, W_hh0, b_ih0,
           b_hh0, W_ih1, W_hh1, b_ih1, b_hh1, dec_w, dec_b):
    half = input.shape[0] // 2
    idx_a = input[:half].reshape(_B)
    idx_b = input[half:].reshape(_B)

    emb, emb2 = _emb_gather(enc_w, idx_a, enc_lyr_w, idx_b)

    gi0 = _input_proj0(emb, emb2, W_ih0, b_ih0)
    y1, h0f = _gru2(gi0.reshape(_SEQ, _BATCH, _NG), hidden[0], hidden[1],
                    W_hh0, b_hh0, W_ih1, b_ih1, W_hh1, b_hh1)

    decoded = _decoder(y1.reshape(_B, _NHID), dec_w, dec_b)
    decoded = decoded.reshape(_SEQ, _BATCH, _NTOKEN)
    hidden_out = jnp.stack([h0f, y1[-1].astype(jnp.float32)], axis=0)
    return decoded, hidden_out
